# baseline (speedup 1.0000x reference)
import jax
import jax.numpy as jnp
from jax import lax
from jax.experimental import pallas as pl
from jax.experimental.pallas import tpu as pltpu
from jax.experimental.pallas import tpu_sc as plsc

B, L, D, M, S = 64, 512, 8, 6, 64
NC, NS, LANES = 2, 16, 16
NW = NC * NS
B_PER_W = B // NW
NCH = S // LANES
NRB = S // LANES
RPB = 16


def _rsqrt(a):
    bits = plsc.bitcast(a, jnp.int32)
    r = plsc.bitcast(jnp.int32(0x5F3759DF) - lax.shift_right_logical(bits, 1),
                     jnp.float32)
    for _ in range(3):
        r = r * (1.5 - 0.5 * a * r * r)
    return r


def _splat(vec, r):
    idx = jnp.full((LANES, 1), r, jnp.int32)
    return jnp.take_along_axis(vec, idx.reshape(LANES), axis=0,
                               mode="promise_in_bounds")


def _body(x_hbm, wpad_hbm, lo_hbm, hi_hbm, rw_hbm, out_hbm,
          lo_v, hi_v, rw_v, xb_v, wb_v, s_v, q_v, h_v, imgbuf, sem):
    wid = lax.axis_index("s") * NC + lax.axis_index("c")
    pltpu.sync_copy(lo_hbm, lo_v)
    pltpu.sync_copy(hi_hbm, hi_v)
    pltpu.sync_copy(rw_hbm, rw_v)
    lane = lax.iota(jnp.int32, LANES)

    def b_loop(bi, _):
        b = wid * B_PER_W + bi
        pltpu.sync_copy(x_hbm.at[b], xb_v)
        pltpu.sync_copy(wpad_hbm.at[b], wb_v)

        def d_loop(d, _):
            dsplat = jnp.full((LANES,), 0, jnp.int32) + d
            raw = []
            for c in range(NCH):
                sl = pl.ds(c * LANES, LANES)
                slo = plsc.load_gather(xb_v, [lo_v[sl] * D + dsplat])
                shi = plsc.load_gather(xb_v, [hi_v[sl] * D + dsplat])
                wr = rw_v[sl]
                raw.append(slo * (1.0 - wr) + shi * wr)
            mnv = jnp.minimum(jnp.minimum(raw[0], raw[1]),
                              jnp.minimum(raw[2], raw[3]))
            mxv = jnp.maximum(jnp.maximum(raw[0], raw[1]),
                              jnp.maximum(raw[2], raw[3]))
            mn = jnp.min(mnv)
            inv = jnp.full((LANES,), 2.0, jnp.float32) / (jnp.max(mxv) - mn + 1e-8)
            sj, qj, hj = [], [], []
            for c in range(NCH):
                sl = pl.ds(c * LANES, LANES)
                s = (raw[c] - mn) * inv - 1.0
                scl = jnp.clip(s, -1.0 + 1e-6, 1.0 - 1e-6)
                aa = 1.0 - scl * scl
                q = aa * _rsqrt(aa)
                h = 0.5 * s
                s_v[sl] = s
                q_v[sl] = q
                h_v[sl] = h
                sj.append(s)
                qj.append(q)
                hj.append(h)
            wv = plsc.load_gather(wb_v, [jnp.minimum(d * M + lane, 63)])
            wv = jnp.where(lane < M, wv, -jnp.inf)
            t = bi * D + d
            p = t % 2
            @pl.when(t >= 2)
            def _():
                pltpu.make_async_copy(
                    imgbuf.at[p], out_hbm.at[b, d], sem).wait()

            def k_loop(k, wv):
                mx = jnp.max(wv)
                mk = jnp.min(jnp.where(wv == mx, lane, 1000))
                wv = jnp.where(lane == mk, -jnp.inf, wv)

                def mk_branch(method):
                    def blk(rb, _):
                        base = rb * RPB
                        sl = pl.ds(base, LANES)
                        sb = s_v[sl]
                        qb = q_v[sl]
                        hb = h_v[sl]
                        for r in range(RPB):
                            i = base + r
                            if method == 0:
                                si = _splat(sb, r)
                                qi = _splat(qb, r)
                                for c in range(NCH):
                                    imgbuf[p, k, i, pl.ds(c * LANES, LANES)] = (
                                        si * sj[c] - qi * qj[c])
                            elif method == 1:
                                si = _splat(sb, r)
                                qi = _splat(qb, r)
                                for c in range(NCH):
                                    imgbuf[p, k, i, pl.ds(c * LANES, LANES)] = (
                                        qi * sj[c] - si * qj[c])
                            elif method == 2:
                                si = _splat(sb, r)
                                for c in range(NCH):
                                    imgbuf[p, k, i, pl.ds(c * LANES, LANES)] = (
                                        0.0 - jnp.abs(sj[c] - si))
                            elif method == 3:
                                si = _splat(sb, r)
                                for c in range(NCH):
                                    imgbuf[p, k, i, pl.ds(c * LANES, LANES)] = (
                                        si * sj[c])
                            elif method == 4:
                                si = _splat(sb, r)
                                for c in range(NCH):
                                    tt = sj[c] - si
                                    imgbuf[p, k, i, pl.ds(c * LANES, LANES)] = (
                                        tt * tt)
                            else:
                                hi_ = _splat(hb, r)
                                for c in range(NCH):
                                    imgbuf[p, k, i, pl.ds(c * LANES, LANES)] = (
                                        hi_ + hj[c])
                        return 0

                    def branch():
                        plsc.parallel_loop(0, S // RPB)(
                            lambda rb: blk(rb, 0) and None)

                    return branch

                lax.switch(mk, [mk_branch(m) for m in range(M)])
                return wv

            lax.fori_loop(0, 3, k_loop, wv)
            pltpu.async_copy(imgbuf.at[p], out_hbm.at[b, d], sem)
            return 0

        lax.fori_loop(0, D, d_loop, 0)
        return 0

    lax.fori_loop(0, B_PER_W, b_loop, 0)
    last = NW * B_PER_W - 1
    for pp in range(2):
        pltpu.make_async_copy(imgbuf.at[pp], out_hbm.at[last, D - 1],
                              sem).wait()


@jax.jit
def kernel(x, ts2img_weights):
    f32, i32 = jnp.float32, jnp.int32
    pos = jnp.linspace(0.0, L - 1.0, S)
    lo = jnp.floor(pos).astype(i32)
    hi = jnp.clip(lo + 1, 0, L - 1)
    rw = (pos - lo.astype(pos.dtype)).astype(f32)
    wpad = jnp.zeros((B, 64), f32).at[:, : D * M].set(
        ts2img_weights.reshape(B, D * M))

    run = pl.kernel(
        _body,
        out_type=jax.ShapeDtypeStruct((B, D, 3, S, S), f32),
        mesh=plsc.VectorSubcoreMesh(
            core_axis_name="c", subcore_axis_name="s",
            num_cores=NC, num_subcores=NS),
        compiler_params=pltpu.CompilerParams(needs_layout_passes=False),
        scratch_types=[
            pltpu.VMEM((S,), i32),
            pltpu.VMEM((S,), i32),
            pltpu.VMEM((S,), f32),
            pltpu.VMEM((L * D,), f32),
            pltpu.VMEM((64,), f32),
            pltpu.VMEM((S + LANES,), f32),
            pltpu.VMEM((S + LANES,), f32),
            pltpu.VMEM((S + LANES,), f32),
            pltpu.VMEM((2, 3, S, S), f32),
            pltpu.SemaphoreType.DMA,
        ],
    )
    return run(x.reshape(B, L * D), wpad, lo, hi, rw)
